# Initial kernel scaffold; baseline (speedup 1.0000x reference)
import functools

import jax
import jax.numpy as jnp
from jax import lax
from jax.experimental import pallas as pl
from jax.experimental.pallas import tpu as pltpu
from jax.experimental.pallas import tpu_sc as plsc

N = 10000
E = 320000
D = 128
S = 8

NC = 2
NS = 16
NW = NC * NS

EC = 80
EDGE_ITERS = E // EC // NW
SEL_CHUNKS = N // 16
SEL_ITERS = -(-SEL_CHUNKS // NW)
ROWS_PER_TILE = N // NS


def _sc_body(with_counts, a_hbm, sm_hbm, src_hbm, dst_hbm, selidx_hbm,
             pagg_hbm, pcnt_hbm, sel_hbm,
             zrow_v, zcnt_v, ones_v, sidx_v, didx_v, rows_v,
             selidx_v, selrows_v, selout_v, acc_sh, cnt_sh, sem):
    c = lax.axis_index("c")
    s = lax.axis_index("s")
    wid = c * NS + s

    z16 = jnp.zeros((16,), jnp.float32)
    o16 = jnp.ones((16,), jnp.float32)

    def zrow_body(r, _):
        for cc in range(D // 16):
            zrow_v[r, pl.ds(cc * 16, 16)] = z16
        return 0
    lax.fori_loop(0, 128, zrow_body, 0)

    def zcnt_body(r, _):
        zcnt_v[r, pl.ds(0, 16)] = z16
        if with_counts:
            ones_v[r, pl.ds(0, 16)] = o16
        return 0
    lax.fori_loop(0, 128, zcnt_body, 0)

    r0 = s * ROWS_PER_TILE
    for k in range(5):
        pltpu.sync_copy(zrow_v.at[pl.ds(0, 125)],
                        acc_sh.at[pl.ds(r0 + k * 125, 125)])
        if with_counts:
            pltpu.sync_copy(zcnt_v.at[pl.ds(0, 125)],
                            cnt_sh.at[pl.ds(r0 + k * 125, 125)])
    plsc.subcore_barrier()

    def edge_body(j, _):
        chunk = j * NW + wid
        base = pl.multiple_of(chunk * EC, EC)
        pltpu.sync_copy(src_hbm.at[pl.ds(base, EC)], sidx_v.at[0])
        pltpu.sync_copy(dst_hbm.at[pl.ds(base, EC)], didx_v.at[0])
        pltpu.async_copy(a_hbm.at[sidx_v.at[0]], rows_v, sem).wait()
        pltpu.sync_copy(rows_v, acc_sh.at[didx_v.at[0]], add=True)
        if with_counts:
            pltpu.sync_copy(ones_v.at[pl.ds(0, EC)],
                            cnt_sh.at[didx_v.at[0]], add=True)
        return 0
    lax.fori_loop(0, EDGE_ITERS, edge_body, 0)

    plsc.subcore_barrier()

    for k in range(5):
        pltpu.sync_copy(acc_sh.at[pl.ds(r0 + k * 125, 125)],
                        pagg_hbm.at[c].at[pl.ds(r0 + k * 125, 125)])
        if with_counts:
            pltpu.sync_copy(cnt_sh.at[pl.ds(r0 + k * 125, 125)],
                            pcnt_hbm.at[c].at[pl.ds(r0 + k * 125, 125)])

    def sel_body(j, _):
        chunk = j * NW + wid

        @pl.when(chunk < SEL_CHUNKS)
        def _():
            ibase = pl.multiple_of(chunk * 128, 128)
            pltpu.sync_copy(selidx_hbm.at[pl.ds(ibase, 128)], selidx_v.at[0])
            pltpu.async_copy(sm_hbm.at[selidx_v.at[0]], selrows_v, sem).wait()

            def row_body(r, _):
                rb = r * S
                for cc in range(D // 16):
                    acc = selrows_v[rb, pl.ds(cc * 16, 16)]
                    for kk in range(1, S):
                        acc = acc + selrows_v[rb + kk, pl.ds(cc * 16, 16)]
                    selout_v[r, pl.ds(cc * 16, 16)] = acc
                return 0
            lax.fori_loop(0, 16, row_body, 0)
            obase = pl.multiple_of(chunk * 16, 16)
            pltpu.sync_copy(selout_v, sel_hbm.at[pl.ds(obase, 16)])
        return 0
    lax.fori_loop(0, SEL_ITERS, sel_body, 0)


def _make_sc_layer(with_counts):
    return pl.kernel(
        functools.partial(_sc_body, with_counts),
        out_type=(
            jax.ShapeDtypeStruct((NC, N, D), jnp.float32),
            jax.ShapeDtypeStruct((NC, N, 16), jnp.float32),
            jax.ShapeDtypeStruct((N, D), jnp.float32),
        ),
        mesh=plsc.VectorSubcoreMesh(core_axis_name="c", subcore_axis_name="s"),
        scratch_types=[
            pltpu.VMEM((128, D), jnp.float32),
            pltpu.VMEM((128, 16), jnp.float32),
            pltpu.VMEM((128, 16), jnp.float32),
            pltpu.VMEM((1, EC), jnp.int32),
            pltpu.VMEM((1, EC), jnp.int32),
            pltpu.VMEM((EC, D), jnp.float32),
            pltpu.VMEM((1, 128), jnp.int32),
            pltpu.VMEM((128, D), jnp.float32),
            pltpu.VMEM((16, D), jnp.float32),
            pltpu.VMEM_SHARED((N, D), jnp.float32),
            pltpu.VMEM_SHARED((N, 16), jnp.float32),
            pltpu.SemaphoreType.DMA,
        ],
    )


_sc_layer0 = _make_sc_layer(True)
_sc_layer1 = _make_sc_layer(False)


BN = 400
GRID = N // BN

_feat = pl.BlockSpec((BN, D), lambda i: (i, 0))
_w = pl.BlockSpec((D, D), lambda i: (0, 0))
_bias = pl.BlockSpec((1, D), lambda i: (0, 0))
_cntb = pl.BlockSpec((BN, 16), lambda i: (i, 0))


def _mm3_body(x_ref, wl_ref, ws_ref, wr_ref, b_ref, a_ref, sm_ref, r_ref):
    xb = x_ref[...]
    a_ref[...] = jnp.dot(xb, wl_ref[...], preferred_element_type=jnp.float32)
    sm_ref[...] = jnp.dot(xb, ws_ref[...] * (1.0 / S),
                          preferred_element_type=jnp.float32)
    r_ref[...] = jnp.dot(xb, wr_ref[...],
                         preferred_element_type=jnp.float32) + b_ref[...]


_tc_front = pl.pallas_call(
    _mm3_body,
    grid=(GRID,),
    in_specs=[_feat, _w, _w, _w, _bias],
    out_specs=[_feat, _feat, _feat],
    out_shape=[jax.ShapeDtypeStruct((N, D), jnp.float32)] * 3,
)


def _mid_body(r0_ref, p0_ref, p1_ref, c0_ref, c1_ref, sel_ref,
              wl_ref, ws_ref, wr_ref, b_ref, a_ref, sm_ref, r_ref):
    cnt = jnp.maximum(c0_ref[:, 0:1] + c1_ref[:, 0:1], 1.0)
    h = r0_ref[...] + (p0_ref[...] + p1_ref[...]) / cnt + sel_ref[...]
    h = jnp.where(h > 0, h, jnp.expm1(h))
    a_ref[...] = jnp.dot(h, wl_ref[...], preferred_element_type=jnp.float32)
    sm_ref[...] = jnp.dot(h, ws_ref[...] * (1.0 / S),
                          preferred_element_type=jnp.float32)
    r_ref[...] = jnp.dot(h, wr_ref[...],
                         preferred_element_type=jnp.float32) + b_ref[...]


_tc_mid = pl.pallas_call(
    _mid_body,
    grid=(GRID,),
    in_specs=[_feat, _feat, _feat, _cntb, _cntb, _feat, _w, _w, _w, _bias],
    out_specs=[_feat, _feat, _feat],
    out_shape=[jax.ShapeDtypeStruct((N, D), jnp.float32)] * 3,
)


def _back_body(r1_ref, p0_ref, p1_ref, c0_ref, c1_ref, sel_ref, o_ref):
    cnt = jnp.maximum(c0_ref[:, 0:1] + c1_ref[:, 0:1], 1.0)
    o_ref[...] = r1_ref[...] + (p0_ref[...] + p1_ref[...]) / cnt + sel_ref[...]


_tc_back = pl.pallas_call(
    _back_body,
    grid=(GRID,),
    in_specs=[_feat, _feat, _feat, _cntb, _cntb, _feat],
    out_specs=_feat,
    out_shape=jax.ShapeDtypeStruct((N, D), jnp.float32),
)


def kernel(x, edge_index, selections, Wl0, Wr0, Ws0, b0, Wl1, Wr1, Ws1, b1):
    src = edge_index[0]
    dst = edge_index[1]
    selflat = selections.reshape(-1)

    a0, sm0, r0 = _tc_front(x, Wl0, Ws0, Wr0, b0.reshape(1, D))
    pagg0, pcnt0, sel0 = _sc_layer0(a0, sm0, src, dst, selflat)
    a1, sm1, r1 = _tc_mid(r0, pagg0[0], pagg0[1], pcnt0[0], pcnt0[1], sel0,
                          Wl1, Ws1, Wr1, b1.reshape(1, D))
    pagg1, _, sel1 = _sc_layer1(a1, sm1, src, dst, selflat)
    out = _tc_back(r1, pagg1[0], pagg1[1], pcnt0[0], pcnt0[1], sel1)
    return out


# speedup vs baseline: 2.0728x; 2.0728x over previous
import jax
import jax.numpy as jnp
from jax import lax
from jax.experimental import pallas as pl
from jax.experimental.pallas import tpu as pltpu
from jax.experimental.pallas import tpu_sc as plsc

N = 10000
E = 320000
D = 128
S = 8

NS = 16
NW = NS

EC = 80
EDGE_ITERS = E // EC // NW
SELC = 8
SEL_CHUNKS = N // SELC
SEL_ITERS = -(-SEL_CHUNKS // NW)
RG = 80
N_RGROUPS = N // RG
RG_LOOPS = -(-N_RGROUPS // NS)

_mesh = plsc.VectorSubcoreMesh(core_axis_name="c", subcore_axis_name="s",
                               num_cores=1)


def _cnt_body(dst_hbm, cnt_hbm, ones_v, didx_v, cnt_sh, sem):
    s = lax.axis_index("s")
    z16 = jnp.zeros((16,), jnp.float32)
    o16 = jnp.ones((16,), jnp.float32)

    def zfill(r, _):
        for cc in range(D // 16):
            ones_v[r, pl.ds(cc * 16, 16)] = z16
        return 0
    lax.fori_loop(0, EC, zfill, 0)

    for j in range(RG_LOOPS):
        g = j * NS + s

        @pl.when(g < N_RGROUPS)
        def _():
            gb = pl.multiple_of(g * RG, RG)
            pltpu.sync_copy(ones_v, cnt_sh.at[pl.ds(gb, RG)])

    def ofill(r, _):
        ones_v[r, pl.ds(0, 16)] = o16
        return 0
    lax.fori_loop(0, EC, ofill, 0)
    plsc.subcore_barrier()

    def cnt_loop(j, _):
        chunk = j * NW + s
        base = pl.multiple_of(chunk * EC, EC)
        pltpu.sync_copy(dst_hbm.at[pl.ds(base, EC)], didx_v.at[0])
        pltpu.sync_copy(ones_v, cnt_sh.at[didx_v.at[0]], add=True)
        return 0
    lax.fori_loop(0, EDGE_ITERS, cnt_loop, 0)

    plsc.subcore_barrier()
    for j in range(RG_LOOPS):
        g = j * NS + s

        @pl.when(g < N_RGROUPS)
        def _():
            gb = pl.multiple_of(g * RG, RG)
            pltpu.sync_copy(cnt_sh.at[pl.ds(gb, RG)], cnt_hbm.at[pl.ds(gb, RG)])


_sc_count = pl.kernel(
    _cnt_body,
    out_type=pltpu.HBM((N, D), jnp.float32),
    mesh=_mesh,
    scratch_types=[
        pltpu.VMEM((EC, D), jnp.float32),
        pltpu.VMEM((1, EC), jnp.int32),
        pltpu.VMEM_SHARED((N, D), jnp.float32),
        pltpu.SemaphoreType.DMA,
    ],
)


def _agg_body(a_hbm, sm_hbm, src_hbm, dst_hbm, selidx_hbm,
              agg_hbm, sel_hbm,
              sidx_v, didx_v, rows_v, selidx_v, selrows_v, selout_v,
              acc_sh, sem):
    s = lax.axis_index("s")
    z16 = jnp.zeros((16,), jnp.float32)

    def zrow_body(r, _):
        for cc in range(D // 16):
            rows_v[r, pl.ds(cc * 16, 16)] = z16
        return 0
    lax.fori_loop(0, EC, zrow_body, 0)

    for j in range(RG_LOOPS):
        g = j * NS + s

        @pl.when(g < N_RGROUPS)
        def _():
            gb = pl.multiple_of(g * RG, RG)
            pltpu.sync_copy(rows_v, acc_sh.at[pl.ds(gb, RG)])
    plsc.subcore_barrier()

    def edge_body(j, _):
        chunk = j * NW + s
        base = pl.multiple_of(chunk * EC, EC)
        pltpu.sync_copy(src_hbm.at[pl.ds(base, EC)], sidx_v.at[0])
        pltpu.sync_copy(dst_hbm.at[pl.ds(base, EC)], didx_v.at[0])
        pltpu.async_copy(a_hbm.at[sidx_v.at[0]], rows_v, sem).wait()
        pltpu.sync_copy(rows_v, acc_sh.at[didx_v.at[0]], add=True)
        return 0
    lax.fori_loop(0, EDGE_ITERS, edge_body, 0)

    plsc.subcore_barrier()

    for j in range(RG_LOOPS):
        g = j * NS + s

        @pl.when(g < N_RGROUPS)
        def _():
            gb = pl.multiple_of(g * RG, RG)
            pltpu.sync_copy(acc_sh.at[pl.ds(gb, RG)], agg_hbm.at[pl.ds(gb, RG)])

    def sel_body(j, _):
        chunk = j * NW + s

        @pl.when(chunk < SEL_CHUNKS)
        def _():
            ibase = pl.multiple_of(chunk * (SELC * S), SELC * S)
            pltpu.sync_copy(selidx_hbm.at[pl.ds(ibase, SELC * S)],
                            selidx_v.at[0])
            pltpu.async_copy(sm_hbm.at[selidx_v.at[0]], selrows_v, sem).wait()

            def row_body(r, _):
                rb = r * S
                for cc in range(D // 16):
                    acc = selrows_v[rb, pl.ds(cc * 16, 16)]
                    for kk in range(1, S):
                        acc = acc + selrows_v[rb + kk, pl.ds(cc * 16, 16)]
                    selout_v[r, pl.ds(cc * 16, 16)] = acc
                return 0
            lax.fori_loop(0, SELC, row_body, 0)
            obase = pl.multiple_of(chunk * SELC, SELC)
            pltpu.sync_copy(selout_v, sel_hbm.at[pl.ds(obase, SELC)])
        return 0
    lax.fori_loop(0, SEL_ITERS, sel_body, 0)


_sc_layer = pl.kernel(
    _agg_body,
    out_type=(
        pltpu.HBM((N, D), jnp.float32),
        pltpu.HBM((N, D), jnp.float32),
    ),
    mesh=_mesh,
    scratch_types=[
        pltpu.VMEM((1, EC), jnp.int32),
        pltpu.VMEM((1, EC), jnp.int32),
        pltpu.VMEM((EC, D), jnp.float32),
        pltpu.VMEM((1, SELC * S), jnp.int32),
        pltpu.VMEM((SELC * S, D), jnp.float32),
        pltpu.VMEM((SELC, D), jnp.float32),
        pltpu.VMEM_SHARED((N, D), jnp.float32),
        pltpu.SemaphoreType.DMA,
    ],
)


BN = 400
GRID = N // BN

_feat = pl.BlockSpec((BN, D), lambda i: (i, 0))
_w = pl.BlockSpec((D, D), lambda i: (0, 0))
_bias = pl.BlockSpec((1, D), lambda i: (0, 0))


def _mm3_body(x_ref, wl_ref, ws_ref, wr_ref, b_ref, a_ref, sm_ref, r_ref):
    xb = x_ref[...]
    a_ref[...] = jnp.dot(xb, wl_ref[...], preferred_element_type=jnp.float32)
    sm_ref[...] = jnp.dot(xb, ws_ref[...] * (1.0 / S),
                          preferred_element_type=jnp.float32)
    r_ref[...] = jnp.dot(xb, wr_ref[...],
                         preferred_element_type=jnp.float32) + b_ref[...]


_tc_front = pl.pallas_call(
    _mm3_body,
    grid=(GRID,),
    in_specs=[_feat, _w, _w, _w, _bias],
    out_specs=[_feat, _feat, _feat],
    out_shape=[jax.ShapeDtypeStruct((N, D), jnp.float32)] * 3,
)


def _mid_body(r0_ref, p0_ref, c0_ref, sel_ref,
              wl_ref, ws_ref, wr_ref, b_ref, a_ref, sm_ref, r_ref):
    cnt = jnp.maximum(c0_ref[:, 0:1], 1.0)
    h = r0_ref[...] + p0_ref[...] / cnt + sel_ref[...]
    h = jnp.where(h > 0, h, jnp.exp(jnp.minimum(h, 0.0)) - 1.0)
    a_ref[...] = jnp.dot(h, wl_ref[...], preferred_element_type=jnp.float32)
    sm_ref[...] = jnp.dot(h, ws_ref[...] * (1.0 / S),
                          preferred_element_type=jnp.float32)
    r_ref[...] = jnp.dot(h, wr_ref[...],
                         preferred_element_type=jnp.float32) + b_ref[...]


_tc_mid = pl.pallas_call(
    _mid_body,
    grid=(GRID,),
    in_specs=[_feat, _feat, _feat, _feat, _w, _w, _w, _bias],
    out_specs=[_feat, _feat, _feat],
    out_shape=[jax.ShapeDtypeStruct((N, D), jnp.float32)] * 3,
)


def _back_body(r1_ref, p0_ref, c0_ref, sel_ref, o_ref):
    cnt = jnp.maximum(c0_ref[:, 0:1], 1.0)
    o_ref[...] = r1_ref[...] + p0_ref[...] / cnt + sel_ref[...]


_tc_back = pl.pallas_call(
    _back_body,
    grid=(GRID,),
    in_specs=[_feat, _feat, _feat, _feat],
    out_specs=_feat,
    out_shape=jax.ShapeDtypeStruct((N, D), jnp.float32),
)


def kernel(x, edge_index, selections, Wl0, Wr0, Ws0, b0, Wl1, Wr1, Ws1, b1):
    src = edge_index[0]
    dst = edge_index[1]
    selflat = selections.reshape(-1)

    cnt0 = _sc_count(dst)
    a0, sm0, r0 = _tc_front(x, Wl0, Ws0, Wr0, b0.reshape(1, D))
    agg0, sel0 = _sc_layer(a0, sm0, src, dst, selflat)
    a1, sm1, r1 = _tc_mid(r0, agg0, cnt0, sel0,
                          Wl1, Ws1, Wr1, b1.reshape(1, D))
    agg1, sel1 = _sc_layer(a1, sm1, src, dst, selflat)
    out = _tc_back(r1, agg1, cnt0, sel1)
    return out
